# Initial kernel scaffold; baseline (speedup 1.0000x reference)
import jax
import jax.numpy as jnp
from jax.experimental import pallas as pl


def kernel(x, edge_index, edge_attr, batch, We1, be1, eps1, W11, b11, g1, bt1, W12, b12, We2, be2, eps2, W21, b21, g2, bt2, W22, b22, lw1, lb1, lw2, lb2):
    raise NotImplementedError("write your pallas kernel here")


# speedup vs baseline: 2.7004x; 2.7004x over previous
import functools

import jax
import jax.numpy as jnp
from jax import lax
from jax.experimental import pallas as pl
from jax.experimental.pallas import tpu as pltpu
from jax.experimental.pallas import tpu_sc as plsc

N = 10000
E = 320000
D_IN = 128
H1 = 256
H2 = 1024
NG = 64
BN_EPS = 1e-5

SC_CORES = 2
SC_TILES = 16
CSTREAM = 128
CK = 1
CH = CSTREAM * CK
NCHUNK = E // CH
NP = 10240
STRIPE = NP // SC_TILES


def _zero_vmem_rows(buf, nrows, ncolgrp):
    zero16 = jnp.zeros((16,), jnp.float32)

    def body(i, _):
        for j in range(ncolgrp):
            buf[i, pl.ds(j * 16, 16)] = zero16
        return 0

    lax.fori_loop(0, nrows, body, 0)


def _init_eeidx(eeidx):
    for j in range(CK):
        for g in range(CSTREAM // 16):
            eeidx[j, pl.ds(g * 16, 16)] = (
                lax.iota(jnp.int32, 16) + (j * CSTREAM + g * 16))


def _edge_chunk(table, ee, src3d, dst3d, agg, srcv, dstv, rows_v, ee_v, eeidx,
                eebase, sem, chunk_id):
    b = chunk_id * CH
    pltpu.sync_copy(src3d.at[chunk_id], srcv)
    pltpu.sync_copy(dst3d.at[chunk_id], dstv)
    for j in range(CK):
        for g in range(CSTREAM // 16):
            sl = pl.ds(g * 16, 16)
            eebase[j, sl] = eeidx[j, sl] + b
    cps = [
        pltpu.async_copy(table.at[srcv.at[j]],
                         rows_v.at[pl.ds(j * CSTREAM, CSTREAM)], sem)
        for j in range(CK)
    ]
    cps += [
        pltpu.async_copy(ee.at[eebase.at[j]],
                         ee_v.at[pl.ds(j * CSTREAM, CSTREAM)], sem)
        for j in range(CK)
    ]
    for cp in cps:
        cp.wait()

    def vrow(i, _):
        for j in range(8):
            sl = pl.ds(j * 16, 16)
            rows_v[i, sl] = jnp.maximum(rows_v[i, sl] + ee_v[i, sl], 0.0)
        return 0

    lax.fori_loop(0, CH, vrow, 0)
    for j in range(CK):
        pltpu.sync_copy(rows_v.at[pl.ds(j * CSTREAM, CSTREAM)],
                        agg.at[dstv.at[j]], add=True)


def _stripe_idx(eeidx, eebase, base):
    for g in range(CSTREAM // 16):
        sl = pl.ds(g * 16, 16)
        eebase[0, sl] = eeidx[0, sl] + base


def _agg_prologue(agg, rows_v, eeidx, eebase, s):
    _zero_vmem_rows(rows_v, CSTREAM, 8)
    r0 = s * STRIPE
    for g in range(STRIPE // CSTREAM):
        _stripe_idx(eeidx, eebase, r0 + g * CSTREAM)
        pltpu.sync_copy(rows_v.at[pl.ds(0, CSTREAM)], agg.at[eebase.at[0]])
    plsc.subcore_barrier()


def _agg_epilogue(agg, out_c, rows_v, eeidx, eebase, sem, s):
    plsc.subcore_barrier()
    r0 = s * STRIPE
    for g in range(STRIPE // CSTREAM):
        _stripe_idx(eeidx, eebase, r0 + g * CSTREAM)
        pltpu.async_copy(agg.at[eebase.at[0]], rows_v.at[pl.ds(0, CSTREAM)],
                         sem).wait()
        pltpu.sync_copy(rows_v.at[pl.ds(0, CSTREAM)], out_c.at[eebase.at[0]])


@functools.cache
def _sc_kernels():
    scratch = [
        pltpu.VMEM((CK, CSTREAM), jnp.int32),
        pltpu.VMEM((CK, CSTREAM), jnp.int32),
        pltpu.VMEM((CH, 128), jnp.float32),
        pltpu.VMEM((CH, 128), jnp.float32),
        pltpu.VMEM((CK, CSTREAM), jnp.int32),
        pltpu.VMEM((CK, CSTREAM), jnp.int32),
        pltpu.VMEM_SHARED((NP, 128), jnp.float32),
        pltpu.SemaphoreType.DMA,
    ]
    mesh = plsc.VectorSubcoreMesh(core_axis_name="c", subcore_axis_name="s")
    kern = functools.partial(
        pl.kernel,
        out_type=jax.ShapeDtypeStruct((2, NP, 128), jnp.float32),
        mesh=mesh,
        scratch_types=scratch,
    )

    @kern
    def sc_conv1(x_hbm, ee_hbm, src3d, dst3d, out_hbm,
                 srcv, dstv, rows_v, ee_v, eeidx, eebase, agg, sem):
        c = lax.axis_index("c")
        s = lax.axis_index("s")
        _init_eeidx(eeidx)
        _agg_prologue(agg, rows_v, eeidx, eebase, s)
        tpc = (E // 2) // CH
        nk = (tpc - s + SC_TILES - 1) // SC_TILES

        def body(i, _):
            chunk_id = c * tpc + s + i * SC_TILES
            _edge_chunk(x_hbm, ee_hbm, src3d, dst3d, agg,
                        srcv, dstv, rows_v, ee_v, eeidx, eebase, sem, chunk_id)
            return 0

        lax.fori_loop(0, nk, body, 0)
        _agg_epilogue(agg, out_hbm.at[c], rows_v, eeidx, eebase, sem, s)

    @kern
    def sc_conv2(h1_hbm, ee2_hbm, src3d, dst3d, out_hbm,
                 srcv, dstv, rows_v, ee_v, eeidx, eebase, agg, sem):
        c = lax.axis_index("c")
        s = lax.axis_index("s")
        _init_eeidx(eeidx)
        _agg_prologue(agg, rows_v, eeidx, eebase, s)
        tpc = E // CH
        nk = (tpc - s + SC_TILES - 1) // SC_TILES

        def body(i, _):
            chunk_id = s + i * SC_TILES
            _edge_chunk(h1_hbm.at[c], ee2_hbm.at[c], src3d, dst3d, agg,
                        srcv, dstv, rows_v, ee_v, eeidx, eebase, sem, chunk_id)
            return 0

        lax.fori_loop(0, nk, body, 0)
        _agg_epilogue(agg, out_hbm.at[c], rows_v, eeidx, eebase, sem, s)

    return sc_conv1, sc_conv2


def _sc_conv1(x, ee1, src3d, dst3d):
    return _sc_kernels()[0](x, ee1, src3d, dst3d)


def _sc_conv2(h1, ee2, src3d, dst3d):
    return _sc_kernels()[1](h1, ee2, src3d, dst3d)


BE = 4000
BN = 1000
NB = N // BN


def _ee_body(ea_ref, w1_ref, b1_ref, w2_ref, b2_ref, o1_ref, o2_ref):
    ea = ea_ref[...]
    o1_ref[...] = jnp.dot(ea, w1_ref[...],
                          preferred_element_type=jnp.float32) + b1_ref[...]
    for h in range(2):
        o2_ref[h] = jnp.dot(ea, w2_ref[h],
                            preferred_element_type=jnp.float32) + b2_ref[h]


def _edge_encoders(edge_attr, We1, be1, We2t, be2t):
    return pl.pallas_call(
        _ee_body,
        grid=(E // BE,),
        in_specs=[
            pl.BlockSpec((BE, 16), lambda i: (i, 0)),
            pl.BlockSpec((16, 128), lambda i: (0, 0)),
            pl.BlockSpec((1, 128), lambda i: (0, 0)),
            pl.BlockSpec((2, 16, 128), lambda i: (0, 0, 0)),
            pl.BlockSpec((2, 1, 128), lambda i: (0, 0, 0)),
        ],
        out_specs=[
            pl.BlockSpec((BE, 128), lambda i: (i, 0)),
            pl.BlockSpec((2, BE, 128), lambda i: (0, i, 0)),
        ],
        out_shape=[
            jax.ShapeDtypeStruct((E, 128), jnp.float32),
            jax.ShapeDtypeStruct((2, E, 128), jnp.float32),
        ],
    )(edge_attr, We1, be1, We2t, be2t)


def _mlp1a_body(x_ref, agg_ref, w_ref, b_ref, eps_ref, h_ref, s_ref):
    i = pl.program_id(0)
    pre = (1.0 + eps_ref[0, 0]) * x_ref[...] + agg_ref[0] + agg_ref[1]
    h = jnp.dot(pre, w_ref[...], preferred_element_type=jnp.float32) + b_ref[...]
    h_ref[...] = h

    @pl.when(i == 0)
    def _():
        s_ref[...] = jnp.zeros_like(s_ref)

    s_ref[0, :] += jnp.sum(h, axis=0)
    s_ref[1, :] += jnp.sum(h * h, axis=0)


def _mlp1a(x, agg1, W11, b11, eps1):
    return pl.pallas_call(
        _mlp1a_body,
        grid=(NB,),
        in_specs=[
            pl.BlockSpec((BN, 128), lambda i: (i, 0)),
            pl.BlockSpec((2, BN, 128), lambda i: (0, i, 0)),
            pl.BlockSpec((128, H1), lambda i: (0, 0)),
            pl.BlockSpec((1, H1), lambda i: (0, 0)),
            pl.BlockSpec((1, 1), lambda i: (0, 0)),
        ],
        out_specs=[
            pl.BlockSpec((BN, H1), lambda i: (i, 0)),
            pl.BlockSpec((2, H1), lambda i: (0, 0)),
        ],
        out_shape=[
            jax.ShapeDtypeStruct((N, H1), jnp.float32),
            jax.ShapeDtypeStruct((2, H1), jnp.float32),
        ],
    )(x, agg1, W11, b11, eps1)


def _mlp1b_body(h_ref, s_ref, g_ref, bt_ref, w_ref, b_ref, o_ref):
    mu = s_ref[0, :] / N
    var = s_ref[1, :] / N - mu * mu
    inv = lax.rsqrt(var + BN_EPS) * g_ref[0, :]
    hn = (h_ref[...] - mu) * inv + bt_ref[0, :]
    hr = jnp.maximum(hn, 0.0)
    y = jnp.dot(hr, w_ref[...], preferred_element_type=jnp.float32) + b_ref[...]
    o_ref[0] = y[:, :128]
    o_ref[1] = y[:, 128:]


def _mlp1b(h1raw, s1, g1, bt1, W12, b12):
    return pl.pallas_call(
        _mlp1b_body,
        grid=(NB,),
        in_specs=[
            pl.BlockSpec((BN, H1), lambda i: (i, 0)),
            pl.BlockSpec((2, H1), lambda i: (0, 0)),
            pl.BlockSpec((1, H1), lambda i: (0, 0)),
            pl.BlockSpec((1, H1), lambda i: (0, 0)),
            pl.BlockSpec((H1, H1), lambda i: (0, 0)),
            pl.BlockSpec((1, H1), lambda i: (0, 0)),
        ],
        out_specs=pl.BlockSpec((2, BN, 128), lambda i: (0, i, 0)),
        out_shape=jax.ShapeDtypeStruct((2, N, 128), jnp.float32),
    )(h1raw, s1, g1, bt1, W12, b12)


def _mlp2a_body(h1_ref, agg_ref, w_ref, b_ref, eps_ref, h_ref, s_ref):
    i = pl.program_id(0)
    hcat = jnp.concatenate([h1_ref[0], h1_ref[1]], axis=1)
    acat = jnp.concatenate([agg_ref[0], agg_ref[1]], axis=1)
    pre = (1.0 + eps_ref[0, 0]) * hcat + acat
    h = jnp.dot(pre, w_ref[...], preferred_element_type=jnp.float32) + b_ref[...]
    h_ref[...] = h

    @pl.when(i == 0)
    def _():
        s_ref[...] = jnp.zeros_like(s_ref)

    s_ref[0, :] += jnp.sum(h, axis=0)
    s_ref[1, :] += jnp.sum(h * h, axis=0)


def _mlp2a(h1, agg2, W21, b21, eps2):
    return pl.pallas_call(
        _mlp2a_body,
        grid=(NB,),
        in_specs=[
            pl.BlockSpec((2, BN, 128), lambda i: (0, i, 0)),
            pl.BlockSpec((2, BN, 128), lambda i: (0, i, 0)),
            pl.BlockSpec((H1, H2), lambda i: (0, 0)),
            pl.BlockSpec((1, H2), lambda i: (0, 0)),
            pl.BlockSpec((1, 1), lambda i: (0, 0)),
        ],
        out_specs=[
            pl.BlockSpec((BN, H2), lambda i: (i, 0)),
            pl.BlockSpec((2, H2), lambda i: (0, 0)),
        ],
        out_shape=[
            jax.ShapeDtypeStruct((N, H2), jnp.float32),
            jax.ShapeDtypeStruct((2, H2), jnp.float32),
        ],
    )(h1, agg2, W21, b21, eps2)


def _mlp2b_body(h_ref, s_ref, g_ref, bt_ref, w_ref, b_ref, batch_ref,
                lw1_ref, lb1_ref, lw2_ref, lb2_ref, o_ref, pool_ref):
    i = pl.program_id(0)
    mu = s_ref[0, :] / N
    var = s_ref[1, :] / N - mu * mu
    inv = lax.rsqrt(var + BN_EPS) * g_ref[0, :]
    hn = (h_ref[...] - mu) * inv + bt_ref[0, :]
    hr = jnp.maximum(hn, 0.0)
    y = jnp.dot(hr, w_ref[...], preferred_element_type=jnp.float32) + b_ref[...]

    @pl.when(i == 0)
    def _():
        pool_ref[...] = jnp.zeros_like(pool_ref)

    b = batch_ref[0, 0, :]
    seg = lax.broadcasted_iota(jnp.int32, (NG, BN), 0)
    onehot = jnp.where(seg == b[None, :], 1.0, 0.0)
    pool_ref[...] += jnp.dot(onehot, y, preferred_element_type=jnp.float32)

    @pl.when(i == NB - 1)
    def _():
        p = pool_ref[...]
        t = jnp.maximum(
            jnp.dot(p, lw1_ref[...], preferred_element_type=jnp.float32)
            + lb1_ref[...], 0.0)
        o_ref[...] = jnp.dot(t, lw2_ref[...],
                             preferred_element_type=jnp.float32) + lb2_ref[...]


def _mlp2b(h2raw, s2, g2, bt2, W22, b22, batch3d, lw1, lb1, lw2, lb2):
    return pl.pallas_call(
        _mlp2b_body,
        grid=(NB,),
        in_specs=[
            pl.BlockSpec((BN, H2), lambda i: (i, 0)),
            pl.BlockSpec((2, H2), lambda i: (0, 0)),
            pl.BlockSpec((1, H2), lambda i: (0, 0)),
            pl.BlockSpec((1, H2), lambda i: (0, 0)),
            pl.BlockSpec((H2, H2), lambda i: (0, 0)),
            pl.BlockSpec((1, H2), lambda i: (0, 0)),
            pl.BlockSpec((1, 1, BN), lambda i: (i, 0, 0)),
            pl.BlockSpec((H2, 128), lambda i: (0, 0)),
            pl.BlockSpec((1, 128), lambda i: (0, 0)),
            pl.BlockSpec((128, 128), lambda i: (0, 0)),
            pl.BlockSpec((1, 128), lambda i: (0, 0)),
        ],
        out_specs=pl.BlockSpec((NG, 128), lambda i: (0, 0)),
        out_shape=jax.ShapeDtypeStruct((NG, 128), jnp.float32),
        scratch_shapes=[pltpu.VMEM((NG, H2), jnp.float32)],
    )(h2raw, s2, g2, bt2, W22, b22, batch3d, lw1, lb1, lw2, lb2)


def kernel(x, edge_index, edge_attr, batch, We1, be1, eps1, W11, b11, g1, bt1,
           W12, b12, We2, be2, eps2, W21, b21, g2, bt2, W22, b22,
           lw1, lb1, lw2, lb2):
    src3d = edge_index[0].reshape(NCHUNK, CK, CSTREAM)
    dst3d = edge_index[1].reshape(NCHUNK, CK, CSTREAM)
    We2t = We2.reshape(16, 2, 128).transpose(1, 0, 2)
    be2t = be2.reshape(2, 1, 128)
    batch3d = batch.reshape(NB, 1, BN)
    eps1r = eps1.reshape(1, 1)
    eps2r = eps2.reshape(1, 1)

    ee1, ee2 = _edge_encoders(edge_attr, We1, be1.reshape(1, 128), We2t, be2t)
    agg1 = _sc_conv1(x, ee1, src3d, dst3d)[:, :N]
    h1raw, s1 = _mlp1a(x, agg1, W11, b11.reshape(1, H1), eps1r)
    h1 = _mlp1b(h1raw, s1, g1.reshape(1, H1), bt1.reshape(1, H1), W12,
                b12.reshape(1, H1))
    agg2 = _sc_conv2(h1, ee2, src3d, dst3d)[:, :N]
    h2raw, s2 = _mlp2a(h1, agg2, W21, b21.reshape(1, H2), eps2r)
    out = _mlp2b(h2raw, s2, g2.reshape(1, H2), bt2.reshape(1, H2), W22,
                 b22.reshape(1, H2), batch3d, lw1, lb1.reshape(1, 128),
                 lw2, lb2.reshape(1, 128))
    return out
